# Initial kernel scaffold; baseline (speedup 1.0000x reference)
import jax
import jax.numpy as jnp
from jax.experimental import pallas as pl


def kernel(x, edge_idx, x_cir, edge_idx_cir, params):
    raise NotImplementedError("write your pallas kernel here")


# speedup vs baseline: 52.5570x; 52.5570x over previous
import functools

import jax
import jax.numpy as jnp
from jax import lax
from jax.experimental import pallas as pl
from jax.experimental.pallas import tpu as pltpu
from jax.experimental.pallas import tpu_sc as plsc

_N_DRUG = 218
_N_CIR = 271
_N = _N_DRUG + _N_CIR
_HID = 128
_HEADS = 4
_NP = 512
_EP_HET = 20480
_EP_CIR = 8192
_PT_HET = _EP_HET // 16
_PT_CIR = _EP_CIR // 16
_CH_HET = _PT_HET // 128
_CH_CIR = _PT_CIR // 128
_CWORDS = _NP * _NP
_PW = _CWORDS // 16


def _count_body(src_h, dst_h, srcc_h, dstc_h, out_het, out_cir,
                src_v, dst_v, flat_v, ones_v, buf_v, c_sh):
    c = lax.axis_index("c")
    s = lax.axis_index("s")

    zeros16 = jnp.zeros((16,), jnp.float32)
    ones16 = jnp.ones((16,), jnp.float32)

    def _zb(i, _):
        buf_v[pl.ds(i * 16, 16)] = zeros16
        return _
    lax.fori_loop(0, _PW // 16, _zb, 0)
    for i in range(8):
        ones_v[pl.ds(i * 16, 16)] = ones16

    pltpu.sync_copy(buf_v, c_sh.at[pl.ds(s * _PW, _PW)])
    plsc.subcore_barrier()

    def _scatter(sh, dh, per_tile, nchunks):
        pltpu.sync_copy(sh.at[pl.ds(s * per_tile, per_tile)],
                        src_v.at[pl.ds(0, per_tile)])
        pltpu.sync_copy(dh.at[pl.ds(s * per_tile, per_tile)],
                        dst_v.at[pl.ds(0, per_tile)])
        for r in range(nchunks):
            def _fb(i, _):
                sv = src_v[pl.ds(r * 128 + i * 16, 16)]
                dv = dst_v[pl.ds(r * 128 + i * 16, 16)]
                flat_v[r, pl.ds(i * 16, 16)] = dv * _NP + sv
                return _
            lax.fori_loop(0, 8, _fb, 0)
        for r in range(nchunks):
            pltpu.sync_copy(ones_v.at[pl.ds(0, 128)],
                            c_sh.at[flat_v.at[r]], add=True)

    @pl.when(c == 0)
    def _():
        _scatter(src_h, dst_h, _PT_HET, _CH_HET)

    @pl.when(c == 1)
    def _():
        _scatter(srcc_h, dstc_h, _PT_CIR, _CH_CIR)

    plsc.subcore_barrier()

    @pl.when(c == 0)
    def _():
        pltpu.sync_copy(c_sh.at[pl.ds(s * _PW, _PW)],
                        out_het.at[pl.ds(s * _PW, _PW)])

    @pl.when(c == 1)
    def _():
        pltpu.sync_copy(c_sh.at[pl.ds(s * _PW, _PW)],
                        out_cir.at[pl.ds(s * _PW, _PW)])


_count_kernel = functools.partial(
    pl.kernel,
    mesh=plsc.VectorSubcoreMesh(core_axis_name="c", subcore_axis_name="s"),
    out_type=[jax.ShapeDtypeStruct((_CWORDS,), jnp.float32),
              jax.ShapeDtypeStruct((_CWORDS,), jnp.float32)],
    scratch_types=[
        pltpu.VMEM((_PT_HET,), jnp.int32),
        pltpu.VMEM((_PT_HET,), jnp.int32),
        pltpu.VMEM((_CH_HET, 128), jnp.int32),
        pltpu.VMEM((128,), jnp.float32),
        pltpu.VMEM((_PW,), jnp.float32),
        pltpu.VMEM_SHARED((_CWORDS,), jnp.float32),
    ],
)(_count_body)


def _dot(a, b, dims=((1,), (1,))):
    return lax.dot_general(
        a, b, (dims, ((), ())),
        precision=lax.Precision.HIGHEST,
        preferred_element_type=jnp.float32)


def _gat_dense(xv, valid, cv, w, asrc, adst, b, wres):
    acc = 0.0
    for hd in range(_HEADS):
        h = _dot(xv, w[hd], dims=((1,), (0,)))
        es = _dot(asrc[hd].reshape(1, _HID), h)
        ed = _dot(h, adst[hd].reshape(1, _HID))
        e = ed + es
        e = jnp.where(e > 0, e, 0.2 * e)
        m = jnp.max(jnp.where(valid, e, -1e30), axis=1, keepdims=True)
        mm = jnp.where(m > -1e29, m, 0.0)
        p = jnp.where(valid, cv * jnp.exp(e - mm), 0.0)
        den = jnp.sum(p, axis=1, keepdims=True)
        alpha = p / (den + 1e-16)
        acc = acc + _dot(alpha, h, dims=((1,), (0,)))
    out = acc * 0.25 + b + _dot(xv, wres, dims=((1,), (0,)))
    return jnp.where(out > 0, out, jnp.exp(jnp.minimum(out, 0.0)) - 1.0)


def _branch(xv, cv, projw, projb, w0, as0, ad0, b0, wr0,
            w1, as1, ad1, b1, wr1, cnnt, cnnb):
    valid = cv > 0.0
    o0 = _dot(xv, projw, dims=((1,), (0,))) + projb
    h1 = _gat_dense(xv, valid, cv, w0, as0, ad0, b0, wr0)
    h2 = _gat_dense(h1, valid, cv, w1, as1, ad1, b1, wr1)
    emb = (_dot(o0, cnnt[0], dims=((1,), (0,)))
           + _dot(h1, cnnt[1], dims=((1,), (0,)))
           + _dot(h2, cnnt[2], dims=((1,), (0,))) + cnnb)
    return emb


def _dense_body(xp_ref, c_ref, xcp_ref, cc_ref,
                projw_ref, projb_ref,
                w0_ref, as0_ref, ad0_ref, b0_ref, wr0_ref,
                w1_ref, as1_ref, ad1_ref, b1_ref, wr1_ref,
                cnnt_ref, cnnb_ref,
                projwc_ref, projbc_ref,
                wc0_ref, asc0_ref, adc0_ref, bc0_ref, wrc0_ref,
                wc1_ref, asc1_ref, adc1_ref, bc1_ref, wrc1_ref,
                cnntc_ref, cnnbc_ref,
                decw_ref, ret_ref, ass_ref):
    emb_het = _branch(xp_ref[...], c_ref[...], projw_ref[...], projb_ref[...],
                      w0_ref[...], as0_ref[...], ad0_ref[...], b0_ref[...],
                      wr0_ref[...], w1_ref[...], as1_ref[...], ad1_ref[...],
                      b1_ref[...], wr1_ref[...], cnnt_ref[...], cnnb_ref[...])
    emb_cir = _branch(xcp_ref[...], cc_ref[...], projwc_ref[...],
                      projbc_ref[...], wc0_ref[...], asc0_ref[...],
                      adc0_ref[...], bc0_ref[...], wrc0_ref[...], wc1_ref[...],
                      asc1_ref[...], adc1_ref[...], bc1_ref[...], wrc1_ref[...],
                      cnntc_ref[...], cnnbc_ref[...])
    drug = lax.slice(emb_het, (0, 0), (_N_DRUG, _HID))
    cir_het = lax.slice(emb_het, (_N_DRUG, 0), (_N, _HID))
    emb_c = lax.slice(emb_cir, (0, 0), (_N_CIR, _HID))
    ass = 0.5 * (cir_het + emb_c)
    t = _dot(drug, decw_ref[...], dims=((1,), (0,)))
    logits = _dot(t, ass)
    ret_ref[...] = 1.0 / (1.0 + jnp.exp(-logits))
    ass_ref[...] = ass


def _pad2(a, rows, cols=None):
    r, c = a.shape
    return jnp.pad(a, ((0, rows - r), (0, (cols or c) - c)))


def kernel(x, edge_idx, x_cir, edge_idx_cir, params):
    p = params

    def _pad_edges(ei, ep):
        n = ei.shape[1]
        src = jnp.concatenate(
            [ei[0].astype(jnp.int32), jnp.zeros((ep - n,), jnp.int32)])
        dst = jnp.concatenate(
            [ei[1].astype(jnp.int32),
             jnp.full((ep - n,), _NP - 1, jnp.int32)])
        return src, dst

    src_h, dst_h = _pad_edges(edge_idx, _EP_HET)
    src_c, dst_c = _pad_edges(edge_idx_cir, _EP_CIR)
    c_het_flat, c_cir_flat = _count_kernel(src_h, dst_h, src_c, dst_c)
    c_het = c_het_flat.reshape(_NP, _NP)
    c_cir = c_cir_flat.reshape(_NP, _NP)

    xp = _pad2(x, _NP, _NP)
    xcp = _pad2(x_cir, _NP, _NP)
    pads = dict(
        projw=_pad2(p['proj_W'], _NP),
        w0=jnp.pad(p['conv0']['W'], ((0, 0), (0, _NP - _N), (0, 0))),
        wr0=_pad2(p['conv0']['W_res'], _NP),
        projwc=_pad2(p['proj_cir_W'], _NP),
        wc0=jnp.pad(p['convc0']['W'], ((0, 0), (0, _NP - _N_CIR), (0, 0))),
        wrc0=_pad2(p['convc0']['W_res'], _NP),
    )
    cnnt = jnp.transpose(p['cnn_het_W'][..., 0], (1, 2, 0))
    cnntc = jnp.transpose(p['cnn_cir_W'][..., 0], (1, 2, 0))
    row = lambda v: v.reshape(1, _HID)

    ret, ass = pl.pallas_call(
        _dense_body,
        out_shape=[jax.ShapeDtypeStruct((_N_DRUG, _N_CIR), jnp.float32),
                   jax.ShapeDtypeStruct((_N_CIR, _HID), jnp.float32)],
    )(xp, c_het, xcp, c_cir,
      pads['projw'], row(p['proj_b']),
      pads['w0'], p['conv0']['a_src'], p['conv0']['a_dst'],
      row(p['conv0']['b']), pads['wr0'],
      p['conv1']['W'], p['conv1']['a_src'], p['conv1']['a_dst'],
      row(p['conv1']['b']), p['conv1']['W_res'],
      cnnt, row(p['cnn_het_b']),
      pads['projwc'], row(p['proj_cir_b']),
      pads['wc0'], p['convc0']['a_src'], p['convc0']['a_dst'],
      row(p['convc0']['b']), pads['wrc0'],
      p['convc1']['W'], p['convc1']['a_src'], p['convc1']['a_dst'],
      row(p['convc1']['b']), p['convc1']['W_res'],
      cnntc, row(p['cnn_cir_b']),
      p['dec_W'])
    return (ret, ass)


# speedup vs baseline: 64.4943x; 1.2271x over previous
import functools

import jax
import jax.numpy as jnp
from jax import lax
from jax.experimental import pallas as pl
from jax.experimental.pallas import tpu as pltpu
from jax.experimental.pallas import tpu_sc as plsc

_N_DRUG = 218
_N_CIR = 271
_N = _N_DRUG + _N_CIR
_HID = 128
_HEADS = 4
_NP = 512
_DUMMY = (_NP - 1) * _NP

_E_HET = 20000
_E_CIR = 8000
_HET_PER = 1248
_HET_LAST = _E_HET - 15 * _HET_PER
_CIR_PER = 504
_CIR_LAST = _E_CIR - 15 * _CIR_PER
_HET_SLOTS = 1280
_CIR_SLOTS = 512

_HET_ROWS = 496
_CIR_ROWS = 288
_HET_PW = _HET_ROWS * _NP // 16
_CIR_PW = _CIR_ROWS * _NP // 16


def _count_body(src_h, dst_h, srcc_h, dstc_h, out_het, out_cir,
                src_v, dst_v, flat_v, ones_v, buf_v, c_sh):
    c = lax.axis_index("c")
    s = lax.axis_index("s")

    zeros16 = jnp.zeros((16,), jnp.float32)
    ones16 = jnp.ones((16,), jnp.float32)
    iota16 = lax.iota(jnp.int32, 16)

    def _zb(i, carry):
        buf_v[pl.ds(i * 16, 16)] = zeros16
        return carry
    lax.fori_loop(0, _HET_PW // 16, _zb, 0)
    for i in range(8):
        ones_v[pl.ds(i * 16, 16)] = ones16

    def _build(sh_h, dh_h, per, last, slots, nchunks, pw):
        pltpu.sync_copy(buf_v.at[pl.ds(0, pw)], c_sh.at[pl.ds(s * pw, pw)])
        plsc.subcore_barrier()

        @pl.when(s < 15)
        def _():
            pltpu.sync_copy(sh_h.at[pl.ds(s * per, per)],
                            src_v.at[pl.ds(0, per)])
            pltpu.sync_copy(dh_h.at[pl.ds(s * per, per)],
                            dst_v.at[pl.ds(0, per)])

        @pl.when(s == 15)
        def _():
            pltpu.sync_copy(sh_h.at[pl.ds(15 * per, last)],
                            src_v.at[pl.ds(0, last)])
            pltpu.sync_copy(dh_h.at[pl.ds(15 * per, last)],
                            dst_v.at[pl.ds(0, last)])

        vc = jnp.where(s == 15, last, per)
        for r in range(nchunks):
            def _fb(i, carry):
                off = r * 128 + i * 16
                sv = src_v[pl.ds(off, 16)]
                dv = dst_v[pl.ds(off, 16)]
                fl = jnp.where(iota16 + off < vc, dv * _NP + sv, _DUMMY)
                flat_v[r, pl.ds(i * 16, 16)] = fl
                return carry
            lax.fori_loop(0, 8, _fb, 0)
        for r in range(nchunks):
            pltpu.sync_copy(ones_v.at[pl.ds(0, 128)],
                            c_sh.at[flat_v.at[r]], add=True)
        plsc.subcore_barrier()

    @pl.when(c == 0)
    def _():
        _build(src_h, dst_h, _HET_PER, _HET_LAST, _HET_SLOTS,
               _HET_SLOTS // 128, _HET_PW)
        pltpu.sync_copy(c_sh.at[pl.ds(s * _HET_PW, _HET_PW)],
                        out_het.at[pl.ds(s * _HET_PW, _HET_PW)])

    @pl.when(c == 1)
    def _():
        _build(srcc_h, dstc_h, _CIR_PER, _CIR_LAST, _CIR_SLOTS,
               _CIR_SLOTS // 128, _CIR_PW)
        pltpu.sync_copy(c_sh.at[pl.ds(s * _CIR_PW, _CIR_PW)],
                        out_cir.at[pl.ds(s * _CIR_PW, _CIR_PW)])


_count_kernel = functools.partial(
    pl.kernel,
    mesh=plsc.VectorSubcoreMesh(core_axis_name="c", subcore_axis_name="s"),
    out_type=[jax.ShapeDtypeStruct((_HET_ROWS * _NP,), jnp.float32),
              jax.ShapeDtypeStruct((_CIR_ROWS * _NP,), jnp.float32)],
    scratch_types=[
        pltpu.VMEM((_HET_SLOTS,), jnp.int32),
        pltpu.VMEM((_HET_SLOTS,), jnp.int32),
        pltpu.VMEM((_HET_SLOTS // 128, 128), jnp.int32),
        pltpu.VMEM((128,), jnp.float32),
        pltpu.VMEM((_HET_PW,), jnp.float32),
        pltpu.VMEM_SHARED((_NP * _NP,), jnp.float32),
    ],
)(_count_body)


def _dot(a, b, dims=((1,), (1,))):
    return lax.dot_general(
        a, b, (dims, ((), ())),
        precision=lax.Precision.HIGHEST,
        preferred_element_type=jnp.float32)


def _gat_dense(xv, valid, cv, w, asrc, adst, b, wres):
    acc = 0.0
    for hd in range(_HEADS):
        h = _dot(xv, w[hd], dims=((1,), (0,)))
        es = _dot(asrc[hd].reshape(1, _HID), h)
        ed = _dot(h, adst[hd].reshape(1, _HID))
        e = ed + es
        e = jnp.where(e > 0, e, 0.2 * e)
        m = jnp.max(jnp.where(valid, e, -1e30), axis=1, keepdims=True)
        mm = jnp.where(m > -1e29, m, 0.0)
        p = jnp.where(valid, cv * jnp.exp(e - mm), 0.0)
        den = jnp.sum(p, axis=1, keepdims=True)
        alpha = p / (den + 1e-16)
        acc = acc + _dot(alpha, h, dims=((1,), (0,)))
    out = acc * 0.25 + b.reshape(1, _HID) + _dot(xv, wres, dims=((1,), (0,)))
    return jnp.where(out > 0, out, jnp.exp(jnp.minimum(out, 0.0)) - 1.0)


def _branch(n, xv, cfull, projw, projb, w0, as0, ad0, b0, wr0,
            w1, as1, ad1, b1, wr1, cnn3, cnnb):
    cv = lax.slice(cfull, (0, 0), (n, n))
    valid = cv > 0.0
    o0 = _dot(xv, projw, dims=((1,), (0,))) + projb.reshape(1, _HID)
    h1 = _gat_dense(xv, valid, cv, w0, as0, ad0, b0, wr0)
    h2 = _gat_dense(h1, valid, cv, w1, as1, ad1, b1, wr1)
    emb = (_dot(o0, cnn3[:, 0, :]) + _dot(h1, cnn3[:, 1, :])
           + _dot(h2, cnn3[:, 2, :]) + cnnb.reshape(1, _HID))
    return emb


def _dense_body(x_ref, c_ref, xc_ref, cc_ref,
                projw_ref, projb_ref,
                w0_ref, as0_ref, ad0_ref, b0_ref, wr0_ref,
                w1_ref, as1_ref, ad1_ref, b1_ref, wr1_ref,
                cnn_ref, cnnb_ref,
                projwc_ref, projbc_ref,
                wc0_ref, asc0_ref, adc0_ref, bc0_ref, wrc0_ref,
                wc1_ref, asc1_ref, adc1_ref, bc1_ref, wrc1_ref,
                cnnc_ref, cnnbc_ref,
                decw_ref, ret_ref, ass_ref):
    emb_het = _branch(_N, x_ref[...], c_ref[...], projw_ref[...],
                      projb_ref[...], w0_ref[...], as0_ref[...], ad0_ref[...],
                      b0_ref[...], wr0_ref[...], w1_ref[...], as1_ref[...],
                      ad1_ref[...], b1_ref[...], wr1_ref[...], cnn_ref[...],
                      cnnb_ref[...])
    emb_cir = _branch(_N_CIR, xc_ref[...], cc_ref[...], projwc_ref[...],
                      projbc_ref[...], wc0_ref[...], asc0_ref[...],
                      adc0_ref[...], bc0_ref[...], wrc0_ref[...], wc1_ref[...],
                      asc1_ref[...], adc1_ref[...], bc1_ref[...],
                      wrc1_ref[...], cnnc_ref[...], cnnbc_ref[...])
    drug = lax.slice(emb_het, (0, 0), (_N_DRUG, _HID))
    cir_het = lax.slice(emb_het, (_N_DRUG, 0), (_N, _HID))
    ass = 0.5 * (cir_het + emb_cir)
    t = _dot(drug, decw_ref[...], dims=((1,), (0,)))
    logits = _dot(t, ass)
    ret_ref[...] = 1.0 / (1.0 + jnp.exp(-logits))
    ass_ref[...] = ass


def kernel(x, edge_idx, x_cir, edge_idx_cir, params):
    p = params
    ei = edge_idx.astype(jnp.int32)
    eic = edge_idx_cir.astype(jnp.int32)
    c_het_flat, c_cir_flat = _count_kernel(ei[0], ei[1], eic[0], eic[1])
    c_het = c_het_flat.reshape(_HET_ROWS, _NP)
    c_cir = c_cir_flat.reshape(_CIR_ROWS, _NP)
    cnn3 = p['cnn_het_W'].reshape(_HID, 3, _HID)
    cnn3c = p['cnn_cir_W'].reshape(_HID, 3, _HID)

    ret, ass = pl.pallas_call(
        _dense_body,
        out_shape=[jax.ShapeDtypeStruct((_N_DRUG, _N_CIR), jnp.float32),
                   jax.ShapeDtypeStruct((_N_CIR, _HID), jnp.float32)],
    )(x, c_het, x_cir, c_cir,
      p['proj_W'], p['proj_b'],
      p['conv0']['W'], p['conv0']['a_src'], p['conv0']['a_dst'],
      p['conv0']['b'], p['conv0']['W_res'],
      p['conv1']['W'], p['conv1']['a_src'], p['conv1']['a_dst'],
      p['conv1']['b'], p['conv1']['W_res'],
      cnn3, p['cnn_het_b'],
      p['proj_cir_W'], p['proj_cir_b'],
      p['convc0']['W'], p['convc0']['a_src'], p['convc0']['a_dst'],
      p['convc0']['b'], p['convc0']['W_res'],
      p['convc1']['W'], p['convc1']['a_src'], p['convc1']['a_dst'],
      p['convc1']['b'], p['convc1']['W_res'],
      cnn3c, p['cnn_cir_b'],
      p['dec_W'])
    return (ret, ass)


# speedup vs baseline: 92.8254x; 1.4393x over previous
import functools

import jax
import jax.numpy as jnp
from jax import lax
from jax.experimental import pallas as pl
from jax.experimental.pallas import tpu as pltpu
from jax.experimental.pallas import tpu_sc as plsc

_N_DRUG = 218
_N_CIR = 271
_N = _N_DRUG + _N_CIR
_HID = 128
_HEADS = 4
_NP = 512
_DUMMY = (_NP - 1) * _NP

_E_HET = 20000
_E_CIR = 8000
_HET_PER = 1248
_HET_LAST = _E_HET - 15 * _HET_PER
_CIR_PER = 504
_CIR_LAST = _E_CIR - 15 * _CIR_PER
_HET_SLOTS = 1280
_CIR_SLOTS = 512

_HET_ROWS = 496
_CIR_ROWS = 288
_HET_PW = _HET_ROWS * _NP // 16
_CIR_PW = _CIR_ROWS * _NP // 16


def _count_body(src_h, dst_h, srcc_h, dstc_h, out_het, out_cir,
                src_v, dst_v, flat_v, ones_v, buf_v, c_sh):
    c = lax.axis_index("c")
    s = lax.axis_index("s")

    zeros16 = jnp.zeros((16,), jnp.float32)
    ones16 = jnp.ones((16,), jnp.float32)
    iota16 = lax.iota(jnp.int32, 16)

    def _zb(i, carry):
        buf_v[pl.ds(i * 16, 16)] = zeros16
        return carry
    lax.fori_loop(0, _HET_PW // 16, _zb, 0)
    for i in range(8):
        ones_v[pl.ds(i * 16, 16)] = ones16

    def _build(sh_h, dh_h, per, last, slots, nchunks, pw):
        pltpu.sync_copy(buf_v.at[pl.ds(0, pw)], c_sh.at[pl.ds(s * pw, pw)])
        plsc.subcore_barrier()

        @pl.when(s < 15)
        def _():
            pltpu.sync_copy(sh_h.at[pl.ds(s * per, per)],
                            src_v.at[pl.ds(0, per)])
            pltpu.sync_copy(dh_h.at[pl.ds(s * per, per)],
                            dst_v.at[pl.ds(0, per)])

        @pl.when(s == 15)
        def _():
            pltpu.sync_copy(sh_h.at[pl.ds(15 * per, last)],
                            src_v.at[pl.ds(0, last)])
            pltpu.sync_copy(dh_h.at[pl.ds(15 * per, last)],
                            dst_v.at[pl.ds(0, last)])

        vc = jnp.where(s == 15, last, per)
        for r in range(nchunks):
            def _fb(i, carry):
                off = r * 128 + i * 16
                sv = src_v[pl.ds(off, 16)]
                dv = dst_v[pl.ds(off, 16)]
                fl = jnp.where(iota16 + off < vc, dv * _NP + sv, _DUMMY)
                flat_v[r, pl.ds(i * 16, 16)] = fl
                return carry
            lax.fori_loop(0, 8, _fb, 0)
        for r in range(nchunks):
            pltpu.sync_copy(ones_v.at[pl.ds(0, 128)],
                            c_sh.at[flat_v.at[r]], add=True)
        plsc.subcore_barrier()

    @pl.when(c == 0)
    def _():
        _build(src_h, dst_h, _HET_PER, _HET_LAST, _HET_SLOTS,
               _HET_SLOTS // 128, _HET_PW)
        pltpu.sync_copy(c_sh.at[pl.ds(s * _HET_PW, _HET_PW)],
                        out_het.at[pl.ds(s * _HET_PW, _HET_PW)])

    @pl.when(c == 1)
    def _():
        _build(srcc_h, dstc_h, _CIR_PER, _CIR_LAST, _CIR_SLOTS,
               _CIR_SLOTS // 128, _CIR_PW)
        pltpu.sync_copy(c_sh.at[pl.ds(s * _CIR_PW, _CIR_PW)],
                        out_cir.at[pl.ds(s * _CIR_PW, _CIR_PW)])


_count_kernel = functools.partial(
    pl.kernel,
    mesh=plsc.VectorSubcoreMesh(core_axis_name="c", subcore_axis_name="s"),
    out_type=[jax.ShapeDtypeStruct((_HET_ROWS * _NP,), jnp.float32),
              jax.ShapeDtypeStruct((_CIR_ROWS * _NP,), jnp.float32)],
    scratch_types=[
        pltpu.VMEM((_HET_SLOTS,), jnp.int32),
        pltpu.VMEM((_HET_SLOTS,), jnp.int32),
        pltpu.VMEM((_HET_SLOTS // 128, 128), jnp.int32),
        pltpu.VMEM((128,), jnp.float32),
        pltpu.VMEM((_HET_PW,), jnp.float32),
        pltpu.VMEM_SHARED((_NP * _NP,), jnp.float32),
    ],
)(_count_body)


def _dot(a, b, dims=((1,), (1,))):
    return lax.dot_general(
        a, b, (dims, ((), ())),
        precision=lax.Precision.DEFAULT,
        preferred_element_type=jnp.float32)


def _gat_dense(xv, valid, cv, w, asrc, adst, b, wres):
    acc = 0.0
    for hd in range(_HEADS):
        h = _dot(xv, w[hd], dims=((1,), (0,)))
        es = _dot(asrc[hd].reshape(1, _HID), h)
        ed = _dot(h, adst[hd].reshape(1, _HID))
        e = ed + es
        e = jnp.where(e > 0, e, 0.2 * e)
        m = jnp.max(jnp.where(valid, e, -1e30), axis=1, keepdims=True)
        mm = jnp.where(m > -1e29, m, 0.0)
        p = jnp.where(valid, cv * jnp.exp(e - mm), 0.0)
        den = jnp.sum(p, axis=1, keepdims=True)
        alpha = p / (den + 1e-16)
        acc = acc + _dot(alpha, h, dims=((1,), (0,)))
    out = acc * 0.25 + b.reshape(1, _HID) + _dot(xv, wres, dims=((1,), (0,)))
    return jnp.where(out > 0, out, jnp.exp(jnp.minimum(out, 0.0)) - 1.0)


def _branch(n, xv, cfull, projw, projb, w0, as0, ad0, b0, wr0,
            w1, as1, ad1, b1, wr1, cnn3, cnnb):
    cv = lax.slice(cfull, (0, 0), (n, n))
    valid = cv > 0.0
    o0 = _dot(xv, projw, dims=((1,), (0,))) + projb.reshape(1, _HID)
    h1 = _gat_dense(xv, valid, cv, w0, as0, ad0, b0, wr0)
    h2 = _gat_dense(h1, valid, cv, w1, as1, ad1, b1, wr1)
    emb = (_dot(o0, cnn3[:, 0, :]) + _dot(h1, cnn3[:, 1, :])
           + _dot(h2, cnn3[:, 2, :]) + cnnb.reshape(1, _HID))
    return emb


def _dense_body(x_ref, c_ref, xc_ref, cc_ref,
                projw_ref, projb_ref,
                w0_ref, as0_ref, ad0_ref, b0_ref, wr0_ref,
                w1_ref, as1_ref, ad1_ref, b1_ref, wr1_ref,
                cnn_ref, cnnb_ref,
                projwc_ref, projbc_ref,
                wc0_ref, asc0_ref, adc0_ref, bc0_ref, wrc0_ref,
                wc1_ref, asc1_ref, adc1_ref, bc1_ref, wrc1_ref,
                cnnc_ref, cnnbc_ref,
                decw_ref, ret_ref, ass_ref):
    emb_het = _branch(_N, x_ref[...], c_ref[...], projw_ref[...],
                      projb_ref[...], w0_ref[...], as0_ref[...], ad0_ref[...],
                      b0_ref[...], wr0_ref[...], w1_ref[...], as1_ref[...],
                      ad1_ref[...], b1_ref[...], wr1_ref[...], cnn_ref[...],
                      cnnb_ref[...])
    emb_cir = _branch(_N_CIR, xc_ref[...], cc_ref[...], projwc_ref[...],
                      projbc_ref[...], wc0_ref[...], asc0_ref[...],
                      adc0_ref[...], bc0_ref[...], wrc0_ref[...], wc1_ref[...],
                      asc1_ref[...], adc1_ref[...], bc1_ref[...],
                      wrc1_ref[...], cnnc_ref[...], cnnbc_ref[...])
    drug = lax.slice(emb_het, (0, 0), (_N_DRUG, _HID))
    cir_het = lax.slice(emb_het, (_N_DRUG, 0), (_N, _HID))
    ass = 0.5 * (cir_het + emb_cir)
    t = _dot(drug, decw_ref[...], dims=((1,), (0,)))
    logits = _dot(t, ass)
    ret_ref[...] = 1.0 / (1.0 + jnp.exp(-logits))
    ass_ref[...] = ass


def kernel(x, edge_idx, x_cir, edge_idx_cir, params):
    p = params
    ei = edge_idx.astype(jnp.int32)
    eic = edge_idx_cir.astype(jnp.int32)
    c_het_flat, c_cir_flat = _count_kernel(ei[0], ei[1], eic[0], eic[1])
    c_het = c_het_flat.reshape(_HET_ROWS, _NP)
    c_cir = c_cir_flat.reshape(_CIR_ROWS, _NP)
    cnn3 = p['cnn_het_W'].reshape(_HID, 3, _HID)
    cnn3c = p['cnn_cir_W'].reshape(_HID, 3, _HID)

    ret, ass = pl.pallas_call(
        _dense_body,
        out_shape=[jax.ShapeDtypeStruct((_N_DRUG, _N_CIR), jnp.float32),
                   jax.ShapeDtypeStruct((_N_CIR, _HID), jnp.float32)],
    )(x, c_het, x_cir, c_cir,
      p['proj_W'], p['proj_b'],
      p['conv0']['W'], p['conv0']['a_src'], p['conv0']['a_dst'],
      p['conv0']['b'], p['conv0']['W_res'],
      p['conv1']['W'], p['conv1']['a_src'], p['conv1']['a_dst'],
      p['conv1']['b'], p['conv1']['W_res'],
      cnn3, p['cnn_het_b'],
      p['proj_cir_W'], p['proj_cir_b'],
      p['convc0']['W'], p['convc0']['a_src'], p['convc0']['a_dst'],
      p['convc0']['b'], p['convc0']['W_res'],
      p['convc1']['W'], p['convc1']['a_src'], p['convc1']['a_dst'],
      p['convc1']['b'], p['convc1']['W_res'],
      cnn3c, p['cnn_cir_b'],
      p['dec_W'])
    return (ret, ass)
